# Initial kernel scaffold; baseline (speedup 1.0000x reference)
import jax
import jax.numpy as jnp
from jax.experimental import pallas as pl


def kernel(x, w_in, w_out, bin_weights):
    raise NotImplementedError("write your pallas kernel here")


# speedup vs baseline: 1.3267x; 1.3267x over previous
import math
import functools

import numpy as np
import jax
import jax.numpy as jnp
from jax.experimental import pallas as pl
from jax.experimental.pallas import tpu as pltpu

_B, _CIN, _CMID, _H, _W = 8, 256, 16, 128, 128
_NA, _RW, _OVR, _EPS = 8, 8, 1.5, 1e-8
_NR = (_H // 2) // _RW + 1
_HW = _H * _W
_PREC = jax.lax.Precision.HIGHEST


@functools.lru_cache(maxsize=1)
def _consts():
    n = _H
    k = np.arange(n)
    F = np.exp(-2j * np.pi * np.outer(k, k) / n) / np.sqrt(n)
    Fs = np.roll(F, -n // 2, axis=0)
    ar = np.ascontiguousarray(Fs.real.astype(np.float32))
    ai = np.ascontiguousarray(Fs.imag.astype(np.float32))

    cy, cx = _H // 2, _W // 2
    yy = (np.arange(_H, dtype=np.float32) - cy)[:, None]
    xx = (np.arange(_W, dtype=np.float32) - cx)[None, :]
    r = np.sqrt(yy * yy + xx * xx)
    theta = (np.arctan2(yy, xx) + math.pi) % math.pi
    ridx = np.clip(np.floor(r / _RW).astype(np.int32), 0, _NR - 1)
    delta = math.pi / _NA
    half = _OVR * delta / 2.0
    centers = (np.arange(_NA, dtype=np.float32) * delta + delta / 2.0)[:, None, None]
    dist = np.abs(theta[None] - centers)
    w = np.clip(1.0 - dist / half, 0.0, None) * (dist < half)
    aw = w / (w.sum(axis=0, keepdims=True) + _EPS)
    basis = np.zeros((_NA * _NR, _HW), dtype=np.float32)
    for a in range(_NA):
        for rr in range(_NR):
            basis[a * _NR + rr] = (aw[a] * (ridx == rr)).reshape(_HW)
    return ar, ai, ar.T.copy(), ai.T.copy(), basis


def _gain_body(bw_ref, basis_ref, gain_ref):
    g = jax.lax.dot_general(bw_ref[...], basis_ref[...],
                            (((1,), (0,)), ((), ())), precision=_PREC)
    gain_ref[...] = g.reshape(_CMID, _H, _W)


def _main_body(x_ref, win_ref, wout_ref, gain_ref, ar_ref, ai_ref,
               art_ref, ait_ref, out_ref, y_s):
    j = pl.program_id(1)

    @pl.when(j == 0)
    def _():
        x2 = x_ref[0]
        proj = jax.lax.dot_general(win_ref[...], x2,
                                   (((1,), (0,)), ((), ())), precision=_PREC)
        proj3 = proj.reshape(_CMID, _H, _W)
        ar = ar_ref[...]
        ai = ai_ref[...]
        art = art_ref[...]
        ait = ait_ref[...]

        def dot(a, b):
            return jax.lax.dot_general(a, b, (((1,), (0,)), ((), ())),
                                       precision=_PREC)

        ys = []
        for m in range(_CMID):
            X = proj3[m]
            U = dot(ar, X)
            V = dot(ai, X)
            Sre = dot(U, art) - dot(V, ait)
            Sim = dot(U, ait) + dot(V, art)
            G = gain_ref[m]
            Ere = G * Sre
            Eim = G * Sim
            Cre = dot(art, Ere) + dot(ait, Eim)
            Cim = dot(art, Eim) - dot(ait, Ere)
            ys.append(dot(Cre, ar) + dot(Cim, ai))
        y_s[...] = jnp.stack(ys).reshape(_CMID, _HW)

    enh = jax.lax.dot_general(wout_ref[...], y_s[...],
                              (((1,), (0,)), ((), ())), precision=_PREC)
    out_ref[0] = x_ref[0, pl.ds(j * 64, 64)] + enh


def kernel(x, w_in, w_out, bin_weights):
    ar, ai, art, ait = (jnp.asarray(a) for a in _consts()[:4])
    basis = jnp.asarray(_consts()[4])
    bw2 = bin_weights.reshape(_CMID, _NA * _NR)

    gain3 = pl.pallas_call(
        _gain_body,
        out_shape=jax.ShapeDtypeStruct((_CMID, _H, _W), jnp.float32),
        name="afe_gain",
    )(bw2, basis)

    x2d = x.reshape(_B, _CIN, _HW)
    out = pl.pallas_call(
        _main_body,
        grid=(_B, 4),
        in_specs=[
            pl.BlockSpec((1, _CIN, _HW), lambda b, j: (b, 0, 0)),
            pl.BlockSpec((_CMID, _CIN), lambda b, j: (0, 0)),
            pl.BlockSpec((64, _CMID), lambda b, j: (j, 0)),
            pl.BlockSpec((_CMID, _H, _W), lambda b, j: (0, 0, 0)),
            pl.BlockSpec((_H, _H), lambda b, j: (0, 0)),
            pl.BlockSpec((_H, _H), lambda b, j: (0, 0)),
            pl.BlockSpec((_H, _H), lambda b, j: (0, 0)),
            pl.BlockSpec((_H, _H), lambda b, j: (0, 0)),
        ],
        out_specs=pl.BlockSpec((1, 64, _HW), lambda b, j: (b, j, 0)),
        out_shape=jax.ShapeDtypeStruct((_B, _CIN, _HW), jnp.float32),
        scratch_shapes=[pltpu.VMEM((_CMID, _HW), jnp.float32)],
        compiler_params=pltpu.CompilerParams(
            dimension_semantics=("parallel", "arbitrary"),
            vmem_limit_bytes=100 * 1024 * 1024,
        ),
        name="afe_main",
    )(x2d, w_in, w_out, gain3, ar, ai, art, ait)
    return out.reshape(_B, _CIN, _H, _W)


# speedup vs baseline: 1.9840x; 1.4954x over previous
import math
import functools

import numpy as np
import jax
import jax.numpy as jnp
from jax.experimental import pallas as pl
from jax.experimental.pallas import tpu as pltpu

_B, _CIN, _CMID, _H, _W = 8, 256, 16, 128, 128
_NA, _RW, _OVR, _EPS = 8, 8, 1.5, 1e-8
_NR = (_H // 2) // _RW + 1
_HW = _H * _W
_PREC = jax.lax.Precision.DEFAULT


@functools.lru_cache(maxsize=1)
def _consts():
    n = _H
    k = np.arange(n)
    F = np.exp(-2j * np.pi * np.outer(k, k) / n) / np.sqrt(n)
    Fs = np.roll(F, -n // 2, axis=0)
    ar = np.ascontiguousarray(Fs.real.astype(np.float32))
    ai = np.ascontiguousarray(Fs.imag.astype(np.float32))

    cy, cx = _H // 2, _W // 2
    yy = (np.arange(_H, dtype=np.float32) - cy)[:, None]
    xx = (np.arange(_W, dtype=np.float32) - cx)[None, :]
    r = np.sqrt(yy * yy + xx * xx)
    theta = (np.arctan2(yy, xx) + math.pi) % math.pi
    ridx = np.clip(np.floor(r / _RW).astype(np.int32), 0, _NR - 1)
    delta = math.pi / _NA
    half = _OVR * delta / 2.0
    centers = (np.arange(_NA, dtype=np.float32) * delta + delta / 2.0)[:, None, None]
    dist = np.abs(theta[None] - centers)
    w = np.clip(1.0 - dist / half, 0.0, None) * (dist < half)
    aw = w / (w.sum(axis=0, keepdims=True) + _EPS)
    basis = np.zeros((_NA * _NR, _HW), dtype=np.float32)
    for a in range(_NA):
        for rr in range(_NR):
            basis[a * _NR + rr] = (aw[a] * (ridx == rr)).reshape(_HW)
    return ar, ai, ar.T.copy(), ai.T.copy(), basis


def _gain_body(bw_ref, basis_ref, gain_ref):
    g = jax.lax.dot_general(bw_ref[...], basis_ref[...],
                            (((1,), (0,)), ((), ())), precision=_PREC)
    gain_ref[...] = g.reshape(_CMID, _H, _W)


def _main_body(x_ref, win_ref, wout_ref, gain_ref, ar_ref, ai_ref,
               art_ref, ait_ref, out_ref, y_s):
    j = pl.program_id(1)

    @pl.when(j == 0)
    def _():
        x2 = x_ref[0]
        proj = jax.lax.dot_general(win_ref[...], x2,
                                   (((1,), (0,)), ((), ())), precision=_PREC)
        proj3 = proj.reshape(_CMID, _H, _W)
        ar = ar_ref[...]
        ai = ai_ref[...]
        art = art_ref[...]
        ait = ait_ref[...]

        def dot(a, b):
            return jax.lax.dot_general(a, b, (((1,), (0,)), ((), ())),
                                       precision=_PREC)

        ys = []
        for m in range(_CMID):
            X = proj3[m]
            U = dot(ar, X)
            V = dot(ai, X)
            Sre = dot(U, art) - dot(V, ait)
            Sim = dot(U, ait) + dot(V, art)
            G = gain_ref[m]
            Ere = G * Sre
            Eim = G * Sim
            Cre = dot(art, Ere) + dot(ait, Eim)
            Cim = dot(art, Eim) - dot(ait, Ere)
            ys.append(dot(Cre, ar) + dot(Cim, ai))
        y_s[...] = jnp.stack(ys).reshape(_CMID, _HW)

    enh = jax.lax.dot_general(wout_ref[...], y_s[...],
                              (((1,), (0,)), ((), ())), precision=_PREC)
    out_ref[0] = x_ref[0, pl.ds(j * 64, 64)] + enh


def kernel(x, w_in, w_out, bin_weights):
    ar, ai, art, ait = (jnp.asarray(a) for a in _consts()[:4])
    basis = jnp.asarray(_consts()[4])
    bw2 = bin_weights.reshape(_CMID, _NA * _NR)

    gain3 = pl.pallas_call(
        _gain_body,
        out_shape=jax.ShapeDtypeStruct((_CMID, _H, _W), jnp.float32),
        name="afe_gain",
    )(bw2, basis)

    x2d = x.reshape(_B, _CIN, _HW)
    out = pl.pallas_call(
        _main_body,
        grid=(_B, 4),
        in_specs=[
            pl.BlockSpec((1, _CIN, _HW), lambda b, j: (b, 0, 0)),
            pl.BlockSpec((_CMID, _CIN), lambda b, j: (0, 0)),
            pl.BlockSpec((64, _CMID), lambda b, j: (j, 0)),
            pl.BlockSpec((_CMID, _H, _W), lambda b, j: (0, 0, 0)),
            pl.BlockSpec((_H, _H), lambda b, j: (0, 0)),
            pl.BlockSpec((_H, _H), lambda b, j: (0, 0)),
            pl.BlockSpec((_H, _H), lambda b, j: (0, 0)),
            pl.BlockSpec((_H, _H), lambda b, j: (0, 0)),
        ],
        out_specs=pl.BlockSpec((1, 64, _HW), lambda b, j: (b, j, 0)),
        out_shape=jax.ShapeDtypeStruct((_B, _CIN, _HW), jnp.float32),
        scratch_shapes=[pltpu.VMEM((_CMID, _HW), jnp.float32)],
        compiler_params=pltpu.CompilerParams(
            dimension_semantics=("parallel", "arbitrary"),
            vmem_limit_bytes=100 * 1024 * 1024,
        ),
        name="afe_main",
    )(x2d, w_in, w_out, gain3, ar, ai, art, ait)
    return out.reshape(_B, _CIN, _H, _W)


# speedup vs baseline: 4.3286x; 2.1818x over previous
import math
import functools

import numpy as np
import jax
import jax.numpy as jnp
from jax.experimental import pallas as pl
from jax.experimental.pallas import tpu as pltpu

_B, _CIN, _CMID, _H, _W = 8, 256, 16, 128, 128
_NA, _RW, _OVR, _EPS = 8, 8, 1.5, 1e-8
_NR = (_H // 2) // _RW + 1
_HW = _H * _W
_PREC = jax.lax.Precision.DEFAULT
_CO = 64


@functools.lru_cache(maxsize=1)
def _consts():
    n = _H
    k = np.arange(n)
    F = np.exp(-2j * np.pi * np.outer(k, k) / n) / np.sqrt(n)
    Fs = np.roll(F, -n // 2, axis=0)
    ar = np.ascontiguousarray(Fs.real.astype(np.float32))
    ai = np.ascontiguousarray(Fs.imag.astype(np.float32))

    cy, cx = _H // 2, _W // 2
    yy = (np.arange(_H, dtype=np.float32) - cy)[:, None]
    xx = (np.arange(_W, dtype=np.float32) - cx)[None, :]
    r = np.sqrt(yy * yy + xx * xx)
    theta = (np.arctan2(yy, xx) + math.pi) % math.pi
    ridx = np.clip(np.floor(r / _RW).astype(np.int32), 0, _NR - 1)
    delta = math.pi / _NA
    half = _OVR * delta / 2.0
    centers = (np.arange(_NA, dtype=np.float32) * delta + delta / 2.0)[:, None, None]
    dist = np.abs(theta[None] - centers)
    w = np.clip(1.0 - dist / half, 0.0, None) * (dist < half)
    aw = w / (w.sum(axis=0, keepdims=True) + _EPS)
    basis = np.zeros((_NA * _NR, _HW), dtype=np.float32)
    for a in range(_NA):
        for rr in range(_NR):
            basis[a * _NR + rr] = (aw[a] * (ridx == rr)).reshape(_HW)

    def _rep_rows(nbig, nsmall):
        s = np.zeros((nbig, nsmall), dtype=np.float32)
        s[np.arange(nbig), np.arange(nbig) // 8] = 1.0
        return s

    def _rep_cols(nsmall, nbig):
        return _rep_rows(nbig, nsmall).T.copy()

    srow_in = _rep_rows(_CMID * 8, _CMID)
    scol_in = _rep_cols(_CIN, _CIN * 8)
    srow_out = _rep_rows(_CO * 8, _CO)
    scol_out = _rep_cols(_CMID, _CMID * 8)
    return ar, ai, basis, srow_in, scol_in, srow_out, scol_out


def _diag8(nrow, ncol):
    r = jax.lax.broadcasted_iota(jnp.int32, (nrow, ncol), 0)
    c = jax.lax.broadcasted_iota(jnp.int32, (nrow, ncol), 1)
    return ((r % 8) == (c % 8)).astype(jnp.float32)


def _dot(a, b):
    return jax.lax.dot_general(a, b, (((1,), (0,)), ((), ())), precision=_PREC)


def _gain_body(bw_ref, basis_ref, gain_ref):
    g = _dot(bw_ref[...], basis_ref[...])
    gain_ref[...] = g.reshape(_CMID, _H, _W)


def _main_body(x_ref, win_ref, wout_ref, gain_ref, ar_ref, ai_ref,
               art_ref, ait_ref, sri_ref, sci_ref, sro_ref, sco_ref,
               out_ref, proj_s, y_s):
    j = pl.program_id(1)
    nh = _H // 8

    @pl.when(j == 0)
    def _():
        wk_in = _dot(_dot(sri_ref[...], win_ref[...]), sci_ref[...])
        wk_in = wk_in * _diag8(_CMID * 8, _CIN * 8)
        for g in range(nh):
            xg = x_ref[0, :, pl.ds(g * 8, 8), :].reshape(_CIN * 8, _W)
            pg = _dot(wk_in, xg)
            proj_s[:, pl.ds(g * 8, 8), :] = pg.reshape(_CMID, 8, _W)

        ar = ar_ref[...]
        ai = ai_ref[...]
        art = art_ref[...]
        ait = ait_ref[...]
        for m in range(_CMID):
            X = proj_s[m]
            U = _dot(ar, X)
            V = _dot(ai, X)
            Sre = _dot(U, art) - _dot(V, ait)
            Sim = _dot(U, ait) + _dot(V, art)
            G = gain_ref[m]
            Ere = G * Sre
            Eim = G * Sim
            Cre = _dot(art, Ere) + _dot(ait, Eim)
            Cim = _dot(art, Eim) - _dot(ait, Ere)
            y_s[m] = _dot(Cre, ar) + _dot(Cim, ai)

    wk_out = _dot(_dot(sro_ref[...], wout_ref[...]), sco_ref[...])
    wk_out = wk_out * _diag8(_CO * 8, _CMID * 8)
    for g in range(nh):
        yg = y_s[:, pl.ds(g * 8, 8), :].reshape(_CMID * 8, _W)
        enh = _dot(wk_out, yg).reshape(_CO, 8, _W)
        xres = x_ref[0, pl.ds(j * _CO, _CO), pl.ds(g * 8, 8), :]
        out_ref[0, :, pl.ds(g * 8, 8), :] = xres + enh


def kernel(x, w_in, w_out, bin_weights):
    ar, ai, basis, sri, sci, sro, sco = (jnp.asarray(a) for a in _consts())
    bw2 = bin_weights.reshape(_CMID, _NA * _NR)

    gain3 = pl.pallas_call(
        _gain_body,
        out_shape=jax.ShapeDtypeStruct((_CMID, _H, _W), jnp.float32),
        name="afe_gain",
    )(bw2, basis)

    out = pl.pallas_call(
        _main_body,
        grid=(_B, _CIN // _CO),
        in_specs=[
            pl.BlockSpec((1, _CIN, _H, _W), lambda b, j: (b, 0, 0, 0)),
            pl.BlockSpec((_CMID, _CIN), lambda b, j: (0, 0)),
            pl.BlockSpec((_CO, _CMID), lambda b, j: (j, 0)),
            pl.BlockSpec((_CMID, _H, _W), lambda b, j: (0, 0, 0)),
            pl.BlockSpec((_H, _H), lambda b, j: (0, 0)),
            pl.BlockSpec((_H, _H), lambda b, j: (0, 0)),
            pl.BlockSpec((_H, _H), lambda b, j: (0, 0)),
            pl.BlockSpec((_H, _H), lambda b, j: (0, 0)),
            pl.BlockSpec((_CMID * 8, _CMID), lambda b, j: (0, 0)),
            pl.BlockSpec((_CIN, _CIN * 8), lambda b, j: (0, 0)),
            pl.BlockSpec((_CO * 8, _CO), lambda b, j: (0, 0)),
            pl.BlockSpec((_CMID, _CMID * 8), lambda b, j: (0, 0)),
        ],
        out_specs=pl.BlockSpec((1, _CO, _H, _W), lambda b, j: (b, j, 0, 0)),
        out_shape=jax.ShapeDtypeStruct((_B, _CIN, _H, _W), jnp.float32),
        scratch_shapes=[pltpu.VMEM((_CMID, _H, _W), jnp.float32),
                        pltpu.VMEM((_CMID, _H, _W), jnp.float32)],
        compiler_params=pltpu.CompilerParams(
            dimension_semantics=("parallel", "arbitrary"),
            vmem_limit_bytes=56 * 1024 * 1024,
        ),
        name="afe_main",
    )(x, w_in, w_out, gain3, ar, ai, ar.T, ai.T, sri, sci, sro, sco)
    return out


# speedup vs baseline: 5.1656x; 1.1934x over previous
import math
import functools

import numpy as np
import jax
import jax.numpy as jnp
from jax.experimental import pallas as pl
from jax.experimental.pallas import tpu as pltpu

_B, _CIN, _CMID, _H, _W = 8, 256, 16, 128, 128
_NA, _RW, _OVR, _EPS = 8, 8, 1.5, 1e-8
_NR = (_H // 2) // _RW + 1
_HW = _H * _W
_CO = 64
_BF = jnp.bfloat16


@functools.lru_cache(maxsize=1)
def _consts():
    n = _H
    k = np.arange(n)
    F = np.exp(-2j * np.pi * np.outer(k, k) / n) / np.sqrt(n)
    Fs = np.roll(F, -n // 2, axis=0)
    ar = np.ascontiguousarray(Fs.real.astype(np.float32))
    ai = np.ascontiguousarray(Fs.imag.astype(np.float32))

    cy, cx = _H // 2, _W // 2
    yy = (np.arange(_H, dtype=np.float32) - cy)[:, None]
    xx = (np.arange(_W, dtype=np.float32) - cx)[None, :]
    r = np.sqrt(yy * yy + xx * xx)
    theta = (np.arctan2(yy, xx) + math.pi) % math.pi
    ridx = np.clip(np.floor(r / _RW).astype(np.int32), 0, _NR - 1)
    delta = math.pi / _NA
    half = _OVR * delta / 2.0
    centers = (np.arange(_NA, dtype=np.float32) * delta + delta / 2.0)[:, None, None]
    dist = np.abs(theta[None] - centers)
    w = np.clip(1.0 - dist / half, 0.0, None) * (dist < half)
    aw = w / (w.sum(axis=0, keepdims=True) + _EPS)
    basis = np.zeros((_NA * _NR, _HW), dtype=np.float32)
    for a in range(_NA):
        for rr in range(_NR):
            basis[a * _NR + rr] = (aw[a] * (ridx == rr)).reshape(_HW)

    def _rep_rows(nbig, nsmall):
        s = np.zeros((nbig, nsmall), dtype=np.float32)
        s[np.arange(nbig), np.arange(nbig) // 8] = 1.0
        return s

    srow_in = _rep_rows(_CMID * 8, _CMID)
    scol_in = _rep_rows(_CIN * 8, _CIN).T.copy()
    srow_out = _rep_rows(_CO * 8, _CO)
    scol_out = _rep_rows(_CMID * 8, _CMID).T.copy()
    return ar, ai, basis, srow_in, scol_in, srow_out, scol_out


def _diag8(nrow, ncol):
    r = jax.lax.broadcasted_iota(jnp.int32, (nrow, ncol), 0)
    c = jax.lax.broadcasted_iota(jnp.int32, (nrow, ncol), 1)
    return ((r % 8) == (c % 8)).astype(jnp.float32)


def _dot(a, b):
    return jax.lax.dot_general(a, b, (((1,), (0,)), ((), ())),
                               preferred_element_type=jnp.float32)


def _bdot(a, b):
    return jax.lax.dot_general(a.astype(_BF), b.astype(_BF),
                               (((1,), (0,)), ((), ())),
                               preferred_element_type=jnp.float32)


def _gain_body(bw_ref, basis_ref, gain_ref):
    g = _dot(bw_ref[...], basis_ref[...])
    gain_ref[...] = g.reshape(_CMID, _H, _W)


def _main_body(x_ref, win_ref, wout_ref, gain_ref, ar_ref, ai_ref,
               art_ref, ait_ref, sri_ref, sci_ref, sro_ref, sco_ref,
               out_ref, proj_s, y_s):
    j = pl.program_id(1)
    nh = _H // 8

    @pl.when(j == 0)
    def _():
        wk_in = _dot(_dot(sri_ref[...], win_ref[...]), sci_ref[...])
        wk_in = (wk_in * _diag8(_CMID * 8, _CIN * 8)).astype(_BF)
        for g in range(0, nh, 2):
            xg = jnp.concatenate(
                [x_ref[0, :, pl.ds(g * 8, 8), :].reshape(_CIN * 8, _W),
                 x_ref[0, :, pl.ds((g + 1) * 8, 8), :].reshape(_CIN * 8, _W)],
                axis=1)
            pg = _bdot(wk_in, xg)
            proj_s[:, pl.ds(g * 8, 8), :] = pg[:, :_W].reshape(_CMID, 8, _W)
            proj_s[:, pl.ds((g + 1) * 8, 8), :] = pg[:, _W:].reshape(_CMID, 8, _W)

        ar = ar_ref[...]
        ai = ai_ref[...]
        art = art_ref[...]
        ait = ait_ref[...]
        for m in range(0, _CMID, 2):
            Xp = jnp.concatenate([proj_s[m], proj_s[m + 1]], axis=1)
            U = _bdot(ar, Xp)
            V = _bdot(ai, Xp)
            Us = jnp.concatenate([U[:, :_W], U[:, _W:]], axis=0)
            Vs = jnp.concatenate([V[:, :_W], V[:, _W:]], axis=0)
            Sre = _bdot(Us, art) - _bdot(Vs, ait)
            Sim = _bdot(Us, ait) + _bdot(Vs, art)
            G = jnp.concatenate([gain_ref[m], gain_ref[m + 1]], axis=0)
            Ere = G * Sre
            Eim = G * Sim
            Erep = jnp.concatenate([Ere[:_H], Ere[_H:]], axis=1)
            Eimp = jnp.concatenate([Eim[:_H], Eim[_H:]], axis=1)
            Cre = _bdot(art, Erep) + _bdot(ait, Eimp)
            Cim = _bdot(art, Eimp) - _bdot(ait, Erep)
            Crs = jnp.concatenate([Cre[:, :_W], Cre[:, _W:]], axis=0)
            Cis = jnp.concatenate([Cim[:, :_W], Cim[:, _W:]], axis=0)
            Y = _bdot(Crs, ar) + _bdot(Cis, ai)
            y_s[m] = Y[:_H]
            y_s[m + 1] = Y[_H:]

    wk_out = _dot(_dot(sro_ref[...], wout_ref[...]), sco_ref[...])
    wk_out = (wk_out * _diag8(_CO * 8, _CMID * 8)).astype(_BF)
    for g in range(0, nh, 2):
        yg = jnp.concatenate(
            [y_s[:, pl.ds(g * 8, 8), :].reshape(_CMID * 8, _W),
             y_s[:, pl.ds((g + 1) * 8, 8), :].reshape(_CMID * 8, _W)],
            axis=1)
        enh = _bdot(wk_out, yg)
        xres0 = x_ref[0, pl.ds(j * _CO, _CO), pl.ds(g * 8, 8), :]
        xres1 = x_ref[0, pl.ds(j * _CO, _CO), pl.ds((g + 1) * 8, 8), :]
        out_ref[0, :, pl.ds(g * 8, 8), :] = xres0 + enh[:, :_W].reshape(_CO, 8, _W)
        out_ref[0, :, pl.ds((g + 1) * 8, 8), :] = xres1 + enh[:, _W:].reshape(_CO, 8, _W)


def kernel(x, w_in, w_out, bin_weights):
    ar, ai, basis, sri, sci, sro, sco = (jnp.asarray(a) for a in _consts())
    bw2 = bin_weights.reshape(_CMID, _NA * _NR)

    gain3 = pl.pallas_call(
        _gain_body,
        out_shape=jax.ShapeDtypeStruct((_CMID, _H, _W), jnp.float32),
        name="afe_gain",
    )(bw2, basis)

    out = pl.pallas_call(
        _main_body,
        grid=(_B, _CIN // _CO),
        in_specs=[
            pl.BlockSpec((1, _CIN, _H, _W), lambda b, j: (b, 0, 0, 0)),
            pl.BlockSpec((_CMID, _CIN), lambda b, j: (0, 0)),
            pl.BlockSpec((_CO, _CMID), lambda b, j: (j, 0)),
            pl.BlockSpec((_CMID, _H, _W), lambda b, j: (0, 0, 0)),
            pl.BlockSpec((_H, _H), lambda b, j: (0, 0)),
            pl.BlockSpec((_H, _H), lambda b, j: (0, 0)),
            pl.BlockSpec((_H, _H), lambda b, j: (0, 0)),
            pl.BlockSpec((_H, _H), lambda b, j: (0, 0)),
            pl.BlockSpec((_CMID * 8, _CMID), lambda b, j: (0, 0)),
            pl.BlockSpec((_CIN, _CIN * 8), lambda b, j: (0, 0)),
            pl.BlockSpec((_CO * 8, _CO), lambda b, j: (0, 0)),
            pl.BlockSpec((_CMID, _CMID * 8), lambda b, j: (0, 0)),
        ],
        out_specs=pl.BlockSpec((1, _CO, _H, _W), lambda b, j: (b, j, 0, 0)),
        out_shape=jax.ShapeDtypeStruct((_B, _CIN, _H, _W), jnp.float32),
        scratch_shapes=[pltpu.VMEM((_CMID, _H, _W), jnp.float32),
                        pltpu.VMEM((_CMID, _H, _W), jnp.float32)],
        compiler_params=pltpu.CompilerParams(
            dimension_semantics=("parallel", "arbitrary"),
            vmem_limit_bytes=56 * 1024 * 1024,
        ),
        name="afe_main",
    )(x, w_in, w_out, gain3, ar, ai, ar.T, ai.T, sri, sci, sro, sco)
    return out
